# baseline (speedup 1.0000x reference)
import functools

import jax
import jax.numpy as jnp
from jax import lax
from jax.experimental import pallas as pl
from jax.experimental.pallas import tpu as pltpu
from jax.experimental.pallas import tpu_sc as plsc

NC = 2
NS = 16
NW = NC * NS
CHUNK = 128
NBUF = 2


def _dist_chunk(buf_a, buf_b, out_b, ncoord, nconf):
  nhalf = nconf // 16

  @plsc.parallel_loop(0, CHUNK, 1, unroll=4)
  def _(e):
    for h in range(nhalf):
      acc = None
      for k in range(ncoord):
        a = buf_a[e, k, pl.ds(h * 16, 16)]
        b = buf_b[e, k, pl.ds(h * 16, 16)]
        d = a - b
        acc = d * d if acc is None else acc + d * d
      half = acc * 0.5
      i = lax.bitcast_convert_type(acc, jnp.int32)
      i = jnp.int32(0x5F3759DF) - (i >> 1)
      r = lax.bitcast_convert_type(i, jnp.float32)
      r = r * (1.5 - (half * r) * r)
      out_b[e, pl.ds(h * 16, 16)] = acc * r


@functools.partial(
    jax.jit, static_argnames=("n_edges", "n_chunks", "ncoord", "nconf"))
def _sc_distances(table, bonds, *, n_edges, n_chunks, ncoord, nconf):
  mesh = plsc.VectorSubcoreMesh(core_axis_name="c", subcore_axis_name="s")
  last_start = n_edges - CHUNK

  @functools.partial(
      pl.kernel,
      out_type=jax.ShapeDtypeStruct((n_edges, nconf), jnp.float32),
      mesh=mesh,
      compiler_params=pltpu.CompilerParams(
          use_tc_tiling_on_sc=False, needs_layout_passes=False),
      scratch_types=[
          pltpu.VMEM((NBUF, CHUNK, 2), jnp.int32),
          pltpu.VMEM((NBUF * 2, CHUNK), jnp.int32),
          pltpu.VMEM((NBUF, CHUNK, ncoord, nconf), jnp.float32),
          pltpu.VMEM((NBUF, CHUNK, ncoord, nconf), jnp.float32),
          pltpu.VMEM((NBUF, CHUNK, nconf), jnp.float32),
          [pltpu.SemaphoreType.DMA] * NBUF,
          [pltpu.SemaphoreType.DMA] * NBUF,
          [pltpu.SemaphoreType.DMA] * NBUF,
          [pltpu.SemaphoreType.DMA] * NBUF,
      ],
  )
  def run(table_h, bonds_h, out_h, bond_v, idx_st, buf_a, buf_b, out_v,
          sem_p, sem_a, sem_b, sem_o):
    wid = lax.axis_index("s") * NC + lax.axis_index("c")

    def start(j):
      return jnp.minimum((wid * n_chunks + j) * CHUNK, last_start)

    def fire_bonds(j, s):
      pltpu.async_copy(
          bonds_h.at[pl.ds(start(j), CHUNK)], bond_v.at[s], sem_p[s])

    def wait_bonds(j, s):
      pltpu.make_async_copy(
          bonds_h.at[pl.ds(start(j), CHUNK)], bond_v.at[s], sem_p[s]).wait()

    def deint(s):
      bv = bond_v.at[s]
      lanes = lax.iota(jnp.int32, 16)
      col0 = jnp.zeros((16,), jnp.int32)
      col1 = jnp.ones((16,), jnp.int32)
      for half in range(CHUNK // 16):
        rows = half * 16 + lanes
        idx_st[2 * s, pl.ds(half * 16, 16)] = plsc.load_gather(
            bv, [rows, col0])
        idx_st[2 * s + 1, pl.ds(half * 16, 16)] = plsc.load_gather(
            bv, [rows, col1])

    def fire_gather(s):
      pltpu.async_copy(
          table_h.at[idx_st.at[2 * s]], buf_a.at[s], sem_a[s])
      pltpu.async_copy(
          table_h.at[idx_st.at[2 * s + 1]], buf_b.at[s], sem_b[s])

    def wait_gather(s):
      pltpu.make_async_copy(
          table_h.at[idx_st.at[2 * s]], buf_a.at[s], sem_a[s]).wait()
      pltpu.make_async_copy(
          table_h.at[idx_st.at[2 * s + 1]], buf_b.at[s], sem_b[s]).wait()

    def fire_out(j, s):
      pltpu.async_copy(
          out_v.at[s], out_h.at[pl.ds(start(j), CHUNK)], sem_o[s])

    def wait_out(j, s):
      pltpu.make_async_copy(
          out_v.at[s], out_h.at[pl.ds(start(j), CHUNK)], sem_o[s]).wait()

    fire_bonds(0, 0)
    fire_bonds(1, 1)
    wait_bonds(0, 0)
    deint(0)
    fire_gather(0)

    @pl.loop(0, n_chunks, step=NBUF)
    def _(j0):
      for b in range(NBUF):
        j = j0 + b
        nxt = 1 - b

        @pl.when(j + 1 < n_chunks)
        def _():
          wait_bonds(j + 1, nxt)
          deint(nxt)
          fire_gather(nxt)

        @pl.when(j + 2 < n_chunks)
        def _():
          fire_bonds(j + 2, b)

        wait_gather(b)

        @pl.when(j >= NBUF)
        def _():
          wait_out(j, b)

        _dist_chunk(buf_a.at[b], buf_b.at[b], out_v.at[b], ncoord, nconf)
        fire_out(j, b)

    for b in range(NBUF):
      wait_out(n_chunks - NBUF + b, b)

  return run(table, bonds)


def kernel(geoms, bonds):
  n_atoms, ncoord, nconf = geoms.shape
  n_edges = bonds.shape[0]
  if bonds.dtype != jnp.int32:
    bonds = bonds.astype(jnp.int32)

  n_chunks = -(-n_edges // (NW * CHUNK))
  n_chunks += (-n_chunks) % NBUF

  return _sc_distances(
      geoms, bonds, n_edges=n_edges, n_chunks=n_chunks, ncoord=ncoord,
      nconf=nconf)


# speedup vs baseline: 1.3329x; 1.3329x over previous
import functools

import jax
import jax.numpy as jnp
from jax import lax
from jax.experimental import pallas as pl
from jax.experimental.pallas import tpu as pltpu
from jax.experimental.pallas import tpu_sc as plsc

NC = 2
NS = 16
NW = NC * NS
CHUNK = 128
NBUF = 2


def _dist_chunk(buf_a, buf_b, out_b, ncoord, nconf):
  nhalf = nconf // 16

  @plsc.parallel_loop(0, CHUNK, 1, unroll=4)
  def _(e):
    for h in range(nhalf):
      acc = None
      for k in range(ncoord):
        a = buf_a[e, pl.ds(k * nconf + h * 16, 16)]
        b = buf_b[e, pl.ds(k * nconf + h * 16, 16)]
        d = a - b
        acc = d * d if acc is None else acc + d * d
      half = acc * 0.5
      i = lax.bitcast_convert_type(acc, jnp.int32)
      i = jnp.int32(0x5F3759DF) - (i >> 1)
      r = lax.bitcast_convert_type(i, jnp.float32)
      r = r * (1.5 - (half * r) * r)
      out_b[e, pl.ds(h * 16, 16)] = acc * r


@functools.partial(
    jax.jit, static_argnames=("n_edges", "n_chunks", "ncoord", "nconf"))
def _sc_distances(table, bonds_flat, *, n_edges, n_chunks, ncoord, nconf):
  d = ncoord * nconf
  mesh = plsc.VectorSubcoreMesh(core_axis_name="c", subcore_axis_name="s")
  last_start = n_edges - CHUNK
  slab = 2 * n_chunks * CHUNK

  @functools.partial(
      pl.kernel,
      out_type=jax.ShapeDtypeStruct((n_edges, nconf), jnp.float32),
      mesh=mesh,
      compiler_params=pltpu.CompilerParams(
          use_tc_tiling_on_sc=False, needs_layout_passes=False),
      scratch_types=[
          pltpu.VMEM((slab,), jnp.int32),
          pltpu.VMEM((NBUF * 2, CHUNK), jnp.int32),
          pltpu.VMEM((NBUF, CHUNK, d), jnp.float32),
          pltpu.VMEM((NBUF, CHUNK, d), jnp.float32),
          pltpu.VMEM((NBUF, CHUNK, nconf), jnp.float32),
          [pltpu.SemaphoreType.DMA] * NBUF,
          [pltpu.SemaphoreType.DMA] * NBUF,
          [pltpu.SemaphoreType.DMA] * NBUF,
      ],
  )
  def run(table_h, bonds_h, out_h, slab_v, idx_st, buf_a, buf_b, out_v,
          sem_a, sem_b, sem_o):
    wid = lax.axis_index("s") * NC + lax.axis_index("c")
    wbase = wid * n_chunks * CHUNK

    def start(j):
      return jnp.minimum(wbase + j * CHUNK, last_start)

    pltpu.sync_copy(bonds_h.at[pl.ds(2 * wbase, slab)], slab_v)

    def deint(j, s):
      base = 2 * (start(j) - wbase)
      lanes = lax.iota(jnp.int32, 16)
      for half in range(CHUNK // 16):
        flat = base + 2 * (half * 16 + lanes)
        idx_st[2 * s, pl.ds(half * 16, 16)] = plsc.load_gather(
            slab_v, [flat])
        idx_st[2 * s + 1, pl.ds(half * 16, 16)] = plsc.load_gather(
            slab_v, [flat + 1])

    def fire_gather(s):
      pltpu.async_copy(
          table_h.at[idx_st.at[2 * s]], buf_a.at[s], sem_a[s])
      pltpu.async_copy(
          table_h.at[idx_st.at[2 * s + 1]], buf_b.at[s], sem_b[s])

    def wait_gather(s):
      pltpu.make_async_copy(
          table_h.at[idx_st.at[2 * s]], buf_a.at[s], sem_a[s]).wait()
      pltpu.make_async_copy(
          table_h.at[idx_st.at[2 * s + 1]], buf_b.at[s], sem_b[s]).wait()

    def fire_out(j, s):
      pltpu.async_copy(
          out_v.at[s], out_h.at[pl.ds(start(j), CHUNK)], sem_o[s])

    def wait_out(j, s):
      pltpu.make_async_copy(
          out_v.at[s], out_h.at[pl.ds(start(j), CHUNK)], sem_o[s]).wait()

    deint(0, 0)
    fire_gather(0)

    @pl.loop(0, n_chunks, step=NBUF)
    def _(j0):
      for b in range(NBUF):
        j = j0 + b
        nxt = 1 - b

        @pl.when(j + 1 < n_chunks)
        def _():
          deint(j + 1, nxt)
          fire_gather(nxt)

        wait_gather(b)

        @pl.when(j >= NBUF)
        def _():
          wait_out(j, b)

        _dist_chunk(buf_a.at[b], buf_b.at[b], out_v.at[b], ncoord, nconf)
        fire_out(j, b)

    for b in range(NBUF):
      wait_out(n_chunks - NBUF + b, b)

  return run(table, bonds_flat)


def kernel(geoms, bonds):
  n_atoms, ncoord, nconf = geoms.shape
  table = geoms.reshape(n_atoms, ncoord * nconf)
  n_edges = bonds.shape[0]
  bonds = bonds.astype(jnp.int32)

  n_chunks = -(-n_edges // (NW * CHUNK))
  n_chunks += (-n_chunks) % NBUF

  flat_len = 2 * NW * n_chunks * CHUNK
  bonds_flat = jnp.pad(bonds.reshape(-1), (0, flat_len - 2 * n_edges))

  return _sc_distances(
      table, bonds_flat, n_edges=n_edges, n_chunks=n_chunks, ncoord=ncoord,
      nconf=nconf)


# speedup vs baseline: 1.3331x; 1.0001x over previous
import functools

import jax
import jax.numpy as jnp
from jax import lax
from jax.experimental import pallas as pl
from jax.experimental.pallas import tpu as pltpu
from jax.experimental.pallas import tpu_sc as plsc

NC = 2
NS = 16
NW = NC * NS
CHUNK = 128
NBUF = 2


def _dist_chunk(buf_a, buf_b, out_b, ncoord, nconf):
  nhalf = nconf // 16

  @plsc.parallel_loop(0, CHUNK, 1, unroll=4)
  def _(e):
    for h in range(nhalf):
      acc = None
      for k in range(ncoord):
        a = buf_a[e, pl.ds(k * nconf + h * 16, 16)]
        b = buf_b[e, pl.ds(k * nconf + h * 16, 16)]
        d = a - b
        acc = d * d if acc is None else acc + d * d
      half = acc * 0.5
      i = lax.bitcast_convert_type(acc, jnp.int32)
      i = jnp.int32(0x5F3759DF) - (i >> 1)
      r = lax.bitcast_convert_type(i, jnp.float32)
      r = r * (1.5 - (half * r) * r)
      out_b[pl.ds(e * nconf + h * 16, 16)] = acc * r


@functools.partial(
    jax.jit, static_argnames=("n_edges", "n_chunks", "ncoord", "nconf"))
def _sc_distances(table, bonds_flat, *, n_edges, n_chunks, ncoord, nconf):
  d = ncoord * nconf
  mesh = plsc.VectorSubcoreMesh(core_axis_name="c", subcore_axis_name="s")
  last_start = n_edges - CHUNK
  slab = 2 * n_chunks * CHUNK

  @functools.partial(
      pl.kernel,
      out_type=jax.ShapeDtypeStruct((n_edges * nconf,), jnp.float32),
      mesh=mesh,
      compiler_params=pltpu.CompilerParams(
          use_tc_tiling_on_sc=False, needs_layout_passes=False),
      scratch_types=[
          pltpu.VMEM((slab,), jnp.int32),
          pltpu.VMEM((NBUF * 2, CHUNK), jnp.int32),
          pltpu.VMEM((NBUF, CHUNK, d), jnp.float32),
          pltpu.VMEM((NBUF, CHUNK, d), jnp.float32),
          pltpu.VMEM((NBUF, CHUNK * nconf), jnp.float32),
          [pltpu.SemaphoreType.DMA] * NBUF,
          [pltpu.SemaphoreType.DMA] * NBUF,
          [pltpu.SemaphoreType.DMA] * NBUF,
      ],
  )
  def run(table_h, bonds_h, out_h, slab_v, idx_st, buf_a, buf_b, out_v,
          sem_a, sem_b, sem_o):
    wid = lax.axis_index("s") * NC + lax.axis_index("c")
    wbase = wid * n_chunks * CHUNK

    def start(j):
      return jnp.minimum(wbase + j * CHUNK, last_start)

    pltpu.sync_copy(bonds_h.at[pl.ds(2 * wbase, slab)], slab_v)

    def deint(j, s):
      base = 2 * (start(j) - wbase)
      lanes = lax.iota(jnp.int32, 16)
      for half in range(CHUNK // 16):
        flat = base + 2 * (half * 16 + lanes)
        idx_st[2 * s, pl.ds(half * 16, 16)] = plsc.load_gather(
            slab_v, [flat])
        idx_st[2 * s + 1, pl.ds(half * 16, 16)] = plsc.load_gather(
            slab_v, [flat + 1])

    def fire_gather(s):
      pltpu.async_copy(
          table_h.at[idx_st.at[2 * s]], buf_a.at[s], sem_a[s])
      pltpu.async_copy(
          table_h.at[idx_st.at[2 * s + 1]], buf_b.at[s], sem_b[s])

    def wait_gather(s):
      pltpu.make_async_copy(
          table_h.at[idx_st.at[2 * s]], buf_a.at[s], sem_a[s]).wait()
      pltpu.make_async_copy(
          table_h.at[idx_st.at[2 * s + 1]], buf_b.at[s], sem_b[s]).wait()

    def fire_out(j, s):
      pltpu.async_copy(
          out_v.at[s], out_h.at[pl.ds(start(j) * nconf, CHUNK * nconf)],
          sem_o[s])

    def wait_out(j, s):
      pltpu.make_async_copy(
          out_v.at[s], out_h.at[pl.ds(start(j) * nconf, CHUNK * nconf)],
          sem_o[s]).wait()

    deint(0, 0)
    fire_gather(0)

    @pl.loop(0, n_chunks, step=NBUF)
    def _(j0):
      for b in range(NBUF):
        j = j0 + b
        nxt = 1 - b

        @pl.when(j + 1 < n_chunks)
        def _():
          deint(j + 1, nxt)
          fire_gather(nxt)

        wait_gather(b)

        @pl.when(j >= NBUF)
        def _():
          wait_out(j, b)

        _dist_chunk(buf_a.at[b], buf_b.at[b], out_v.at[b], ncoord, nconf)
        fire_out(j, b)

    for b in range(NBUF):
      wait_out(n_chunks - NBUF + b, b)

  return run(table, bonds_flat)


def kernel(geoms, bonds):
  n_atoms, ncoord, nconf = geoms.shape
  table = geoms.reshape(n_atoms, ncoord * nconf)
  n_edges = bonds.shape[0]
  bonds = bonds.astype(jnp.int32)

  n_chunks = -(-n_edges // (NW * CHUNK))
  n_chunks += (-n_chunks) % NBUF

  flat_len = 2 * NW * n_chunks * CHUNK
  bonds_flat = jnp.pad(bonds.reshape(-1), (0, flat_len - 2 * n_edges))

  out = _sc_distances(
      table, bonds_flat, n_edges=n_edges, n_chunks=n_chunks, ncoord=ncoord,
      nconf=nconf)
  return out.reshape(n_edges, nconf)
